# baseline (speedup 1.0000x reference)
import functools

import jax
import jax.numpy as jnp
from jax import lax
from jax.experimental import pallas as pl
from jax.experimental.pallas import tpu as pltpu
from jax.experimental.pallas import tpu_sc as plsc

NUM_TASKS = 100000
NUM_EXPERTS = 128
BATCH = 16384

_NC = 2
_NS = 16
_NW = _NC * _NS
_B_PER_W = BATCH // _NW
_CHUNK = 128
_NCHUNK = _B_PER_W // _CHUNK


def _sc_gather(table, idx3):
    mesh = plsc.VectorSubcoreMesh(core_axis_name="c", subcore_axis_name="s")

    @functools.partial(
        pl.kernel,
        out_type=jax.ShapeDtypeStruct((BATCH, NUM_EXPERTS), jnp.float32),
        mesh=mesh,
        scratch_types=[
            pltpu.VMEM((_NCHUNK, _CHUNK), jnp.int32),
            pltpu.VMEM((_B_PER_W, NUM_EXPERTS), jnp.float32),
            pltpu.SemaphoreType.DMA,
        ],
    )
    def k(table_hbm, idx_hbm, out_hbm, idx_v, rows_v, sem):
        wid = lax.axis_index("s") * _NC + lax.axis_index("c")
        base = wid * _B_PER_W
        pltpu.sync_copy(idx_hbm.at[wid], idx_v)
        copies = []
        for j in range(_NCHUNK):
            copies.append(
                pltpu.async_copy(
                    table_hbm.at[idx_v.at[j]],
                    rows_v.at[pl.ds(j * _CHUNK, _CHUNK), :],
                    sem,
                )
            )
        for c in copies:
            c.wait()
        pltpu.sync_copy(rows_v, out_hbm.at[pl.ds(base, _B_PER_W), :])

    return k(table, idx3)


def _tc_transpose(rows):
    blk = 512

    def body(x_ref, o_ref):
        o_ref[...] = x_ref[...].T

    return pl.pallas_call(
        body,
        grid=(BATCH // blk,),
        in_specs=[pl.BlockSpec((blk, NUM_EXPERTS), lambda i: (i, 0))],
        out_specs=pl.BlockSpec((NUM_EXPERTS, blk), lambda i: (0, i)),
        out_shape=jax.ShapeDtypeStruct((NUM_EXPERTS, BATCH), jnp.float32),
    )(rows)


def kernel(env_index, task_index_to_mask):
    idx = env_index.reshape(_NW, _NCHUNK, _CHUNK).astype(jnp.int32)
    rows = _sc_gather(task_index_to_mask, idx)
    out = _tc_transpose(rows)
    return out[:, :, None]


# speedup vs baseline: 1.3960x; 1.3960x over previous
import functools

import jax
import jax.numpy as jnp
from jax import lax
from jax.experimental import pallas as pl
from jax.experimental.pallas import tpu as pltpu
from jax.experimental.pallas import tpu_sc as plsc

NUM_TASKS = 100000
NUM_EXPERTS = 128
BATCH = 16384

_NC = 2
_NS = 16
_NW = _NC * _NS
_B_PER_W = BATCH // _NW
_CHUNK = 128
_NCHUNK = _B_PER_W // _CHUNK


def _sc_gather(table, idx3):
    mesh = plsc.VectorSubcoreMesh(core_axis_name="c", subcore_axis_name="s")

    @functools.partial(
        pl.kernel,
        out_type=jax.ShapeDtypeStruct((BATCH, NUM_EXPERTS), jnp.float32),
        mesh=mesh,
        scratch_types=[
            pltpu.VMEM((_NCHUNK, _CHUNK), jnp.int32),
            pltpu.VMEM((_B_PER_W, NUM_EXPERTS), jnp.float32),
            pltpu.SemaphoreType.DMA,
        ],
    )
    def k(table_hbm, idx_hbm, out_hbm, idx_v, rows_v, sem):
        wid = lax.axis_index("s") * _NC + lax.axis_index("c")
        base = wid * _B_PER_W
        pltpu.sync_copy(idx_hbm.at[wid], idx_v)
        copies = []
        for j in range(_NCHUNK):
            copies.append(
                pltpu.async_copy(
                    table_hbm.at[idx_v.at[j]],
                    rows_v.at[pl.ds(j * _CHUNK, _CHUNK), :],
                    sem,
                )
            )
        for c in copies:
            c.wait()
        pltpu.sync_copy(rows_v, out_hbm.at[pl.ds(base, _B_PER_W), :])

    return k(table, idx3)


def _tc_transpose(rows):
    blk = 1024

    def body(x_ref, o_ref):
        for j in range(blk // NUM_EXPERTS):
            o_ref[:, j, :] = x_ref[pl.ds(j * NUM_EXPERTS, NUM_EXPERTS), :].T

    return pl.pallas_call(
        body,
        grid=(BATCH // blk,),
        in_specs=[pl.BlockSpec((blk, NUM_EXPERTS), lambda i: (i, 0))],
        out_specs=pl.BlockSpec(
            (NUM_EXPERTS, blk // NUM_EXPERTS, NUM_EXPERTS), lambda i: (0, i, 0)
        ),
        out_shape=jax.ShapeDtypeStruct(
            (NUM_EXPERTS, BATCH // NUM_EXPERTS, NUM_EXPERTS), jnp.float32
        ),
    )(rows)


def kernel(env_index, task_index_to_mask):
    idx = env_index.reshape(_NW, _NCHUNK, _CHUNK).astype(jnp.int32)
    rows = _sc_gather(task_index_to_mask, idx)
    out = _tc_transpose(rows)
    return out.reshape(NUM_EXPERTS, BATCH)[:, :, None]
